# baseline (speedup 1.0000x reference)
import functools

import jax
import jax.numpy as jnp
from jax import lax
from jax.experimental import pallas as pl
from jax.experimental.pallas import tpu as pltpu
from jax.experimental.pallas import tpu_sc as plsc

NC, NS, L = 2, 16, 16
NW = NC * NS
B = 65536
KEPT = 32768
D = 768
BINS = B // NW
CHUNK = 4096
K = 64
HCAP = 3072 + 5 * L
NZS = 8


def _dbl(rows):
    def body(r, _):
        for k in range(D // L):
            x = rows[r, pl.ds(k * L, L)]
            rows[r, pl.ds(k * L, L)] = x + x
        return None

    lax.fori_loop(0, K, body, None)


def _restore_kernel(outputs_hbm, indices_hbm, out_hbm,
                    idx_a, idx_b, hitv, hitj, gjl, grl, zrl, hist_t,
                    rows_a, rows_b, stg_ga, stg_gb, stg_sa, stg_sb, stg_z,
                    sem_ga, sem_gb, sem_sa, sem_sb, sem_z, sem_ia, sem_ib):
    wid = lax.axis_index("s") * NC + lax.axis_index("c")
    lo = wid * BINS
    lane = lax.iota(jnp.int32, L)
    lane0 = lane == 0
    zero16 = jnp.zeros((L,), jnp.int32)
    onehot = jnp.where(lane0, jnp.int32(1), jnp.int32(0))

    def zh(b, _):
        hist_t[pl.ds(b * L, L)] = zero16
        return None

    lax.fori_loop(0, (BINS + L) // L, zh, None)

    NCH = B // CHUNK

    def scan_chunk(buf, ci, carry):
        def vec4(i4, c2):
            nh, nbase = c2
            for u in range(8):
                i = i4 * 8 + u
                v = buf[pl.ds(i * L, L)]
                nbase = nbase + plsc.all_reduce_population_count(v < lo)[0]
                m = plsc.bitcast(v - lo, jnp.uint32) < jnp.uint32(BINS)
                cnt = plsc.all_reduce_population_count(m)[0]
                jv = ci * CHUNK + i * L + lane
                plsc.store_compressed(hitv.at[pl.ds(nh, L)], v, mask=m)
                plsc.store_compressed(hitj.at[pl.ds(nh, L)], jv, mask=m)
                nh = nh + cnt
            return (nh, nbase)

        return lax.fori_loop(0, CHUNK // L // 8, vec4, carry)

    def chunk_pair(cp, carry):
        c0 = cp * 2
        c1 = c0 + 1

        @pl.when(cp > 0)
        def _():
            pltpu.make_async_copy(indices_hbm.at[pl.ds(c0 * CHUNK, CHUNK)],
                                  idx_a, sem_ia).wait()

        pltpu.async_copy(indices_hbm.at[pl.ds(c1 * CHUNK, CHUNK)], idx_b,
                         sem_ib)
        carry = scan_chunk(idx_a, c0, carry)
        pltpu.make_async_copy(indices_hbm.at[pl.ds(c1 * CHUNK, CHUNK)], idx_b,
                              sem_ib).wait()

        @pl.when(cp + 1 < NCH // 2)
        def _():
            pltpu.async_copy(indices_hbm.at[pl.ds((c0 + 2) * CHUNK, CHUNK)],
                             idx_a, sem_ia)

        return scan_chunk(idx_b, c1, carry)

    with jax.named_scope("p1_scan"):
        pltpu.sync_copy(indices_hbm.at[pl.ds(0, CHUNK)], idx_a)
        nh, nbase = lax.fori_loop(0, NCH // 2, chunk_pair,
                                  (jnp.int32(0), jnp.int32(0)))

    hitv[pl.ds(nh, L)] = zero16 + (lo + BINS)
    hitj[pl.ds(nh, L)] = zero16 + KEPT
    nhb = (nh + L - 1) // L

    def hb(tb, _):
        hv = hitv[pl.ds(tb * L, L)]
        for q in range(L):
            b = hv[q] - lo
            tv = hist_t[pl.ds(b, L)]
            hist_t[pl.ds(b, L)] = tv + onehot
        return None

    with jax.named_scope("p2_hist"):
        lax.fori_loop(0, nhb, hb, None)

    def pf(b, carry):
        h = hist_t[pl.ds(b * L, L)]
        c = plsc.cumsum(h)
        hist_t[pl.ds(b * L, L)] = carry + c - h
        return carry + c[L - 1]

    lax.fori_loop(0, BINS // L, pf, nbase)

    def ra(tb, carry):
        nv, nz = carry
        hv = hitv[pl.ds(tb * L, L)]
        hj = hitj[pl.ds(tb * L, L)]
        for q in range(L):
            v = hv[q]
            j = hj[q]
            b = v - lo
            tv = hist_t[pl.ds(b, L)]
            r = tv[0]
            hist_t[pl.ds(b, L)] = tv + onehot
            val = j < KEPT
            live = tb * L + q < nh

            @pl.when(val)
            def _(nv=nv, j=j, r=r):
                og = gjl[pl.ds(nv, L)]
                gjl[pl.ds(nv, L)] = jnp.where(lane0, j, og)
                orr = grl[pl.ds(nv, L)]
                grl[pl.ds(nv, L)] = jnp.where(lane0, r, orr)

            zl = jnp.logical_and(jnp.logical_not(val), live)

            @pl.when(zl)
            def _(nz=nz, r=r):
                oz = zrl[pl.ds(nz, L)]
                zrl[pl.ds(nz, L)] = jnp.where(lane0, r, oz)

            nv = nv + val.astype(jnp.int32)
            nz = nz + zl.astype(jnp.int32)
        return (nv, nz)

    with jax.named_scope("p3_rank"):
        nv, nz = lax.fori_loop(0, nhb, ra, (jnp.int32(0), jnp.int32(0)))

    @pl.when(nv > 0)
    def _():
        jl = zero16 + gjl[pl.ds(nv - 1, L)][0]
        rl = zero16 + grl[pl.ds(nv - 1, L)][0]
        for q in range(K // L):
            gjl[pl.ds(nv + q * L, L)] = jl
            grl[pl.ds(nv + q * L, L)] = rl

    @pl.when(nz > 0)
    def _():
        zl = zero16 + zrl[pl.ds(nz - 1, L)][0]
        for q in range(K // L):
            zrl[pl.ds(nz + q * L, L)] = zl

    ncv = (nv + K - 1) // K
    ncz = (nz + K - 1) // K

    def mvp(g, _):
        c0 = g * 2
        c1 = c0 + 1
        has1 = c1 < ncv

        @pl.when(g > 0)
        def _():
            pltpu.make_async_copy(rows_a, out_hbm.at[stg_sa], sem_sa).wait()

        for q in range(K // L):
            stg_ga[pl.ds(q * L, L)] = gjl[pl.ds(c0 * K + q * L, L)]
        pltpu.async_copy(outputs_hbm.at[stg_ga], rows_a, sem_ga)

        @pl.when(jnp.logical_and(g > 0, 2 * g - 1 < ncv))
        def _():
            pltpu.make_async_copy(rows_b, out_hbm.at[stg_sb], sem_sb).wait()

        @pl.when(has1)
        def _():
            for q in range(K // L):
                stg_gb[pl.ds(q * L, L)] = gjl[pl.ds(c1 * K + q * L, L)]
            pltpu.async_copy(outputs_hbm.at[stg_gb], rows_b, sem_gb)

        pltpu.make_async_copy(outputs_hbm.at[stg_ga], rows_a, sem_ga).wait()
        _dbl(rows_a)
        for q in range(K // L):
            stg_sa[pl.ds(q * L, L)] = grl[pl.ds(c0 * K + q * L, L)]
        pltpu.async_copy(rows_a, out_hbm.at[stg_sa], sem_sa)

        @pl.when(has1)
        def _():
            pltpu.make_async_copy(outputs_hbm.at[stg_gb], rows_b, sem_gb).wait()
            _dbl(rows_b)
            for q in range(K // L):
                stg_sb[pl.ds(q * L, L)] = grl[pl.ds(c1 * K + q * L, L)]
            pltpu.async_copy(rows_b, out_hbm.at[stg_sb], sem_sb)

        return None

    with jax.named_scope("p4_move"):
        lax.fori_loop(0, (ncv + 1) // 2, mvp, None)

        @pl.when(ncv > 0)
        def _():
            pltpu.make_async_copy(rows_a, out_hbm.at[stg_sa], sem_sa).wait()

        @pl.when(jnp.logical_and(ncv >= 2, ncv % 2 == 0))
        def _():
            pltpu.make_async_copy(rows_b, out_hbm.at[stg_sb], sem_sb).wait()

    zf = jnp.zeros((L,), jnp.float32)

    def zb(r, _):
        for k in range(D // L):
            rows_a[r, pl.ds(k * L, L)] = zf
        return None

    lax.fori_loop(0, K, zb, None)

    def mz(g, carry):
        for s in range(NZS):
            c = g * NZS + s

            @pl.when(c < ncz)
            def _(c=c, s=s):
                for q in range(K // L):
                    stg_z[s, pl.ds(q * L, L)] = zrl[pl.ds(c * K + q * L, L)]
                pltpu.async_copy(rows_a, out_hbm.at[stg_z.at[s]], sem_z)

        for s in range(NZS):
            c = g * NZS + s

            @pl.when(c < ncz)
            def _(s=s):
                pltpu.make_async_copy(rows_a, out_hbm.at[stg_z.at[s]],
                                      sem_z).wait()

        return None

    with jax.named_scope("p5_zeros"):
        lax.fori_loop(0, (ncz + NZS - 1) // NZS, mz, None)


def kernel(outputs, indices):
    mesh = plsc.VectorSubcoreMesh(core_axis_name="c", subcore_axis_name="s")
    f = functools.partial(
        pl.kernel,
        out_type=jax.ShapeDtypeStruct((B, D), jnp.float32),
        mesh=mesh,
        compiler_params=pltpu.CompilerParams(needs_layout_passes=False),
        scratch_types=[
            pltpu.VMEM((CHUNK,), jnp.int32),
            pltpu.VMEM((CHUNK,), jnp.int32),
            pltpu.VMEM((HCAP,), jnp.int32),
            pltpu.VMEM((HCAP,), jnp.int32),
            pltpu.VMEM((HCAP,), jnp.int32),
            pltpu.VMEM((HCAP,), jnp.int32),
            pltpu.VMEM((HCAP,), jnp.int32),
            pltpu.VMEM((BINS + L,), jnp.int32),
            pltpu.VMEM((K, D), jnp.float32),
            pltpu.VMEM((K, D), jnp.float32),
            pltpu.VMEM((K,), jnp.int32),
            pltpu.VMEM((K,), jnp.int32),
            pltpu.VMEM((K,), jnp.int32),
            pltpu.VMEM((K,), jnp.int32),
            pltpu.VMEM((NZS, K), jnp.int32),
            pltpu.SemaphoreType.DMA,
            pltpu.SemaphoreType.DMA,
            pltpu.SemaphoreType.DMA,
            pltpu.SemaphoreType.DMA,
            pltpu.SemaphoreType.DMA,
            pltpu.SemaphoreType.DMA,
            pltpu.SemaphoreType.DMA,
        ],
    )(_restore_kernel)
    return f(outputs, indices)


# speedup vs baseline: 1.1697x; 1.1697x over previous
import functools

import jax
import jax.numpy as jnp
from jax import lax
from jax.experimental import pallas as pl
from jax.experimental.pallas import tpu as pltpu
from jax.experimental.pallas import tpu_sc as plsc

NC, NS, L = 2, 16, 16
NW = NC * NS
B = 65536
KEPT = 32768
D = 768
BINS = B // NW
CHUNK = 4096
K = 32
HCAP = 3072 + 5 * L
NZS = 8


def _dbl(rows):
    def body(r, _):
        for k in range(D // L):
            x = rows[r, pl.ds(k * L, L)]
            rows[r, pl.ds(k * L, L)] = x + x
        return None

    lax.fori_loop(0, K, body, None)


def _restore_kernel(outputs_hbm, indices_hbm, out_hbm,
                    idx_a, idx_b, hitv, hitj, gjl, grl, zrl, hist_t,
                    rows_a, rows_b, rows_z, stg_ga, stg_gb, stg_sa, stg_sb,
                    stg_z,
                    sem_ga, sem_gb, sem_sa, sem_sb, sem_z, sem_ia, sem_ib):
    wid = lax.axis_index("s") * NC + lax.axis_index("c")
    lo = wid * BINS
    lane = lax.iota(jnp.int32, L)
    lane0 = lane == 0
    zero16 = jnp.zeros((L,), jnp.int32)
    onehot = jnp.where(lane0, jnp.int32(1), jnp.int32(0))

    def zh(b, _):
        hist_t[pl.ds(b * L, L)] = zero16
        return None

    lax.fori_loop(0, (BINS + L) // L, zh, None)

    NCH = B // CHUNK

    def scan_chunk(buf, ci, carry):
        def vec4(i4, c2):
            nh, nbase = c2
            for u in range(8):
                i = i4 * 8 + u
                v = buf[pl.ds(i * L, L)]
                nbase = nbase + plsc.all_reduce_population_count(v < lo)[0]
                m = plsc.bitcast(v - lo, jnp.uint32) < jnp.uint32(BINS)
                cnt = plsc.all_reduce_population_count(m)[0]
                jv = ci * CHUNK + i * L + lane
                plsc.store_compressed(hitv.at[pl.ds(nh, L)], v, mask=m)
                plsc.store_compressed(hitj.at[pl.ds(nh, L)], jv, mask=m)
                nh = nh + cnt
            return (nh, nbase)

        return lax.fori_loop(0, CHUNK // L // 8, vec4, carry)

    def chunk_pair(cp, carry):
        c0 = cp * 2
        c1 = c0 + 1

        @pl.when(cp > 0)
        def _():
            pltpu.make_async_copy(indices_hbm.at[pl.ds(c0 * CHUNK, CHUNK)],
                                  idx_a, sem_ia).wait()

        pltpu.async_copy(indices_hbm.at[pl.ds(c1 * CHUNK, CHUNK)], idx_b,
                         sem_ib)
        carry = scan_chunk(idx_a, c0, carry)
        pltpu.make_async_copy(indices_hbm.at[pl.ds(c1 * CHUNK, CHUNK)], idx_b,
                              sem_ib).wait()

        @pl.when(cp + 1 < NCH // 2)
        def _():
            pltpu.async_copy(indices_hbm.at[pl.ds((c0 + 2) * CHUNK, CHUNK)],
                             idx_a, sem_ia)

        return scan_chunk(idx_b, c1, carry)

    with jax.named_scope("p1_scan"):
        pltpu.sync_copy(indices_hbm.at[pl.ds(0, CHUNK)], idx_a)
        nh, nbase = lax.fori_loop(0, NCH // 2, chunk_pair,
                                  (jnp.int32(0), jnp.int32(0)))

    hitv[pl.ds(nh, L)] = zero16 + (lo + BINS)
    hitj[pl.ds(nh, L)] = zero16 + KEPT
    nhb = (nh + L - 1) // L

    def hb(tb, _):
        hv = hitv[pl.ds(tb * L, L)]
        for q in range(L):
            b = hv[q] - lo
            tv = hist_t[pl.ds(b, L)]
            hist_t[pl.ds(b, L)] = tv + onehot
        return None

    with jax.named_scope("p2_hist"):
        lax.fori_loop(0, nhb, hb, None)

    def pf(b, carry):
        h = hist_t[pl.ds(b * L, L)]
        c = plsc.cumsum(h)
        hist_t[pl.ds(b * L, L)] = carry + c - h
        return carry + c[L - 1]

    lax.fori_loop(0, BINS // L, pf, nbase)

    def ra(tb, carry):
        nv, nz = carry
        hv = hitv[pl.ds(tb * L, L)]
        hj = hitj[pl.ds(tb * L, L)]
        for q in range(L):
            v = hv[q]
            j = hj[q]
            b = v - lo
            tv = hist_t[pl.ds(b, L)]
            r = tv[0]
            hist_t[pl.ds(b, L)] = tv + onehot
            val = j < KEPT
            live = tb * L + q < nh

            @pl.when(val)
            def _(nv=nv, j=j, r=r):
                og = gjl[pl.ds(nv, L)]
                gjl[pl.ds(nv, L)] = jnp.where(lane0, j, og)
                orr = grl[pl.ds(nv, L)]
                grl[pl.ds(nv, L)] = jnp.where(lane0, r, orr)

            zl = jnp.logical_and(jnp.logical_not(val), live)

            @pl.when(zl)
            def _(nz=nz, r=r):
                oz = zrl[pl.ds(nz, L)]
                zrl[pl.ds(nz, L)] = jnp.where(lane0, r, oz)

            nv = nv + val.astype(jnp.int32)
            nz = nz + zl.astype(jnp.int32)
        return (nv, nz)

    with jax.named_scope("p3_rank"):
        nv, nz = lax.fori_loop(0, nhb, ra, (jnp.int32(0), jnp.int32(0)))

    @pl.when(nv > 0)
    def _():
        jl = zero16 + gjl[pl.ds(nv - 1, L)][0]
        rl = zero16 + grl[pl.ds(nv - 1, L)][0]
        for q in range(K // L):
            gjl[pl.ds(nv + q * L, L)] = jl
            grl[pl.ds(nv + q * L, L)] = rl

    @pl.when(nz > 0)
    def _():
        zl = zero16 + zrl[pl.ds(nz - 1, L)][0]
        for q in range(K // L):
            zrl[pl.ds(nz + q * L, L)] = zl

    ncv = (nv + K - 1) // K
    ncz = (nz + K - 1) // K

    zf = jnp.zeros((L,), jnp.float32)

    def zb(r, _):
        for k in range(D // L):
            rows_z[r, pl.ds(k * L, L)] = zf
        return None

    lax.fori_loop(0, K, zb, None)

    def zfire(z):
        @pl.when(z < ncz)
        def _():
            @pl.when(z >= NZS)
            def _():
                pltpu.make_async_copy(rows_z, out_hbm.at[stg_z.at[0]],
                                      sem_z).wait()

            s = z % NZS
            for q in range(K // L):
                stg_z[s, pl.ds(q * L, L)] = zrl[pl.ds(z * K + q * L, L)]
            pltpu.async_copy(rows_z, out_hbm.at[stg_z.at[s]], sem_z)

    def mvp(g, _):
        c0 = g * 2
        c1 = c0 + 1

        @pl.when(jnp.logical_and(g > 0, c0 - 2 < ncv))
        def _():
            pltpu.make_async_copy(rows_a, out_hbm.at[stg_sa], sem_sa).wait()

        @pl.when(c0 < ncv)
        def _():
            for q in range(K // L):
                stg_ga[pl.ds(q * L, L)] = gjl[pl.ds(c0 * K + q * L, L)]
            pltpu.async_copy(outputs_hbm.at[stg_ga], rows_a, sem_ga)

        zfire(c0)

        @pl.when(jnp.logical_and(g > 0, c0 - 1 < ncv))
        def _():
            pltpu.make_async_copy(rows_b, out_hbm.at[stg_sb], sem_sb).wait()

        @pl.when(c1 < ncv)
        def _():
            for q in range(K // L):
                stg_gb[pl.ds(q * L, L)] = gjl[pl.ds(c1 * K + q * L, L)]
            pltpu.async_copy(outputs_hbm.at[stg_gb], rows_b, sem_gb)

        zfire(c1)

        @pl.when(c0 < ncv)
        def _():
            pltpu.make_async_copy(outputs_hbm.at[stg_ga], rows_a,
                                  sem_ga).wait()
            _dbl(rows_a)
            for q in range(K // L):
                stg_sa[pl.ds(q * L, L)] = grl[pl.ds(c0 * K + q * L, L)]
            pltpu.async_copy(rows_a, out_hbm.at[stg_sa], sem_sa)

        @pl.when(c1 < ncv)
        def _():
            pltpu.make_async_copy(outputs_hbm.at[stg_gb], rows_b,
                                  sem_gb).wait()
            _dbl(rows_b)
            for q in range(K // L):
                stg_sb[pl.ds(q * L, L)] = grl[pl.ds(c1 * K + q * L, L)]
            pltpu.async_copy(rows_b, out_hbm.at[stg_sb], sem_sb)

        return None

    nct = jnp.maximum((ncv + 1) // 2, (ncz + 1) // 2)
    with jax.named_scope("p4_move"):
        lax.fori_loop(0, nct, mvp, None)

        @pl.when(jnp.logical_and(ncv > 0, nct == (ncv + 1) // 2))
        def _():
            pltpu.make_async_copy(rows_a, out_hbm.at[stg_sa], sem_sa).wait()

        @pl.when(jnp.logical_and(ncv >= 2, nct * 2 == ncv))
        def _():
            pltpu.make_async_copy(rows_b, out_hbm.at[stg_sb], sem_sb).wait()

        for s in range(NZS):
            @pl.when(s < jnp.minimum(ncz, NZS))
            def _():
                pltpu.make_async_copy(rows_z, out_hbm.at[stg_z.at[0]],
                                      sem_z).wait()


def kernel(outputs, indices):
    mesh = plsc.VectorSubcoreMesh(core_axis_name="c", subcore_axis_name="s")
    f = functools.partial(
        pl.kernel,
        out_type=jax.ShapeDtypeStruct((B, D), jnp.float32),
        mesh=mesh,
        compiler_params=pltpu.CompilerParams(needs_layout_passes=False),
        scratch_types=[
            pltpu.VMEM((CHUNK,), jnp.int32),
            pltpu.VMEM((CHUNK,), jnp.int32),
            pltpu.VMEM((HCAP,), jnp.int32),
            pltpu.VMEM((HCAP,), jnp.int32),
            pltpu.VMEM((HCAP,), jnp.int32),
            pltpu.VMEM((HCAP,), jnp.int32),
            pltpu.VMEM((HCAP,), jnp.int32),
            pltpu.VMEM((BINS + L,), jnp.int32),
            pltpu.VMEM((K, D), jnp.float32),
            pltpu.VMEM((K, D), jnp.float32),
            pltpu.VMEM((K, D), jnp.float32),
            pltpu.VMEM((K,), jnp.int32),
            pltpu.VMEM((K,), jnp.int32),
            pltpu.VMEM((K,), jnp.int32),
            pltpu.VMEM((K,), jnp.int32),
            pltpu.VMEM((NZS, K), jnp.int32),
            pltpu.SemaphoreType.DMA,
            pltpu.SemaphoreType.DMA,
            pltpu.SemaphoreType.DMA,
            pltpu.SemaphoreType.DMA,
            pltpu.SemaphoreType.DMA,
            pltpu.SemaphoreType.DMA,
            pltpu.SemaphoreType.DMA,
        ],
    )(_restore_kernel)
    return f(outputs, indices)
